# baseline (speedup 1.0000x reference)
import jax
import jax.numpy as jnp
from jax import lax
from jax.experimental import pallas as pl
from jax.experimental.pallas import tpu as pltpu


def _gnn_kernel(adj_a_ref, adj_b_ref, ft_ref, w_ref, ot_ref, st_ref):
    @pl.when(pl.program_id(0) == 0)
    def _():
        st_ref[...] = lax.dot_general(
            w_ref[...],
            ft_ref[...],
            (((0,), (0,)), ((), ())),
            preferred_element_type=jnp.float32,
        ).astype(jnp.bfloat16)

    bm = adj_a_ref.shape[0]
    ot_ref[:, 0:bm] = jnp.maximum(
        lax.dot_general(
            st_ref[...],
            adj_a_ref[...].astype(jnp.bfloat16),
            (((1,), (1,)), ((), ())),
            preferred_element_type=jnp.float32,
        ),
        0.0,
    )
    ot_ref[:, bm : 2 * bm] = jnp.maximum(
        lax.dot_general(
            st_ref[...],
            adj_b_ref[...].astype(jnp.bfloat16),
            (((1,), (1,)), ((), ())),
            preferred_element_type=jnp.float32,
        ),
        0.0,
    )


def kernel(features, adj, W):
    n, d_in = features.shape
    d_out = W.shape[1]
    bm = 256
    grid = (n // (2 * bm),)
    out_t = pl.pallas_call(
        _gnn_kernel,
        grid=grid,
        in_specs=[
            pl.BlockSpec((bm, n), lambda i: (2 * i, 0)),
            pl.BlockSpec((bm, n), lambda i: (2 * i + 1, 0)),
            pl.BlockSpec((d_in, n), lambda i: (0, 0)),
            pl.BlockSpec((d_in, d_out), lambda i: (0, 0)),
        ],
        out_specs=pl.BlockSpec((d_out, 2 * bm), lambda i: (0, i)),
        out_shape=jax.ShapeDtypeStruct((d_out, n), jnp.float32),
        scratch_shapes=[pltpu.VMEM((d_out, n), jnp.bfloat16)],
    )(adj, adj, features.T, W)
    return out_t.T


# speedup vs baseline: 1.0059x; 1.0059x over previous
import jax
import jax.numpy as jnp
from jax import lax
from jax.experimental import pallas as pl
from jax.experimental.pallas import tpu as pltpu


def _gnn_kernel(adj_ref, ft_ref, w_ref, ot_ref, st_ref):
    @pl.when(pl.program_id(0) == 0)
    def _():
        st_ref[...] = lax.dot_general(
            w_ref[...],
            ft_ref[...],
            (((0,), (0,)), ((), ())),
            preferred_element_type=jnp.float32,
        ).astype(jnp.bfloat16)

    ot_ref[...] = jnp.maximum(
        lax.dot_general(
            st_ref[...],
            adj_ref[...].astype(jnp.bfloat16),
            (((1,), (1,)), ((), ())),
            preferred_element_type=jnp.float32,
        ),
        0.0,
    )


def kernel(features, adj, W):
    n, d_in = features.shape
    d_out = W.shape[1]
    bm = 512
    grid = (n // bm,)
    out_t = pl.pallas_call(
        _gnn_kernel,
        grid=grid,
        in_specs=[
            pl.BlockSpec((bm, n), lambda i: (i, 0)),
            pl.BlockSpec((d_in, n), lambda i: (0, 0)),
            pl.BlockSpec((d_in, d_out), lambda i: (0, 0)),
        ],
        out_specs=pl.BlockSpec((d_out, bm), lambda i: (0, i)),
        out_shape=jax.ShapeDtypeStruct((d_out, n), jnp.float32),
        scratch_shapes=[pltpu.VMEM((d_out, n), jnp.bfloat16)],
    )(adj, features.T, W)
    return out_t.T
